# baseline (speedup 1.0000x reference)
import functools

import jax
import jax.numpy as jnp
from jax import lax
from jax.experimental import pallas as pl
from jax.experimental.pallas import tpu as pltpu
from jax.experimental.pallas import tpu_sc as plsc

N = 10000
D = 256
E = 160000
PE = 80000

NC = 2
NS = 16
L = 16
NW = NC * NS

NPAD = 10240
ROWS_PER_TILE = NPAD // NW

SK = 2000
NCH = E // SK
GB = 64
CBUF = 96

CK = 64
PC = PE // CK
CPT = (PC + NW - 1) // NW

_MESH = plsc.VectorSubcoreMesh(
    core_axis_name="c", subcore_axis_name="s", num_cores=NC, num_subcores=NS)


@functools.partial(
    pl.kernel,
    out_type=jax.ShapeDtypeStruct((NPAD, D), jnp.float32),
    mesh=_MESH,
    compiler_params=pltpu.CompilerParams(needs_layout_passes=False),
    scratch_types=[
        pltpu.VMEM((ROWS_PER_TILE, D), jnp.float32),
        pltpu.VMEM((ROWS_PER_TILE + 2 * L, ), jnp.float32),
        pltpu.VMEM((2 * GB, D), jnp.float32),
        pltpu.VMEM((2 * SK,), jnp.int32),
        pltpu.VMEM((2 * SK,), jnp.int32),
        pltpu.VMEM((CBUF,), jnp.int32),
        pltpu.VMEM((CBUF,), jnp.int32),
        pltpu.VMEM((2 * GB,), jnp.int32),
        pltpu.VMEM((2 * GB + L,), jnp.int32),
        pltpu.SemaphoreType.DMA,
        pltpu.SemaphoreType.DMA,
        pltpu.SemaphoreType.DMA,
    ],
)
def _agg_kernel(x_hbm, src_hbm, dst_hbm, out_hbm,
                agg_part, deg1d, rowbuf, srcs_v, dsts_v, csrc, cld, gidx,
                cldsave, sem1, sem2, semg):
    c = lax.axis_index("c")
    s = lax.axis_index("s")
    w = s * NC + c
    my_base = w * ROWS_PER_TILE

    zeros16f = jnp.zeros((L,), jnp.float32)
    zeros16i = jnp.zeros((L,), jnp.int32)
    iota = lax.iota(jnp.int32, L)

    @pl.loop(0, ROWS_PER_TILE)
    def _(r):
        for j in range(D // L):
            agg_part[r, pl.ds(j * L, L)] = zeros16f
    for t in range((ROWS_PER_TILE + 2 * L) // L):
        deg1d[pl.ds(t * L, L)] = zeros16f
    for t in range(CBUF // L):
        csrc[pl.ds(t * L, L)] = zeros16i

    def _start_gather(p):
        pltpu.async_copy(x_hbm.at[gidx.at[pl.ds(p * GB, GB)]],
                         rowbuf.at[pl.ds(p * GB, GB)], semg)

    def _consume(p, n):
        pltpu.make_async_copy(x_hbm.at[gidx.at[pl.ds(p * GB, GB)]],
                              rowbuf.at[pl.ds(p * GB, GB)], semg).wait()

        def edge(e, carry):
            ld = cldsave[pl.ds(p * GB + e, L)][0]
            dbase = ld & ~(L - 1)
            onehot = jnp.where(iota == ld - dbase, 1.0, 0.0)
            plsc.addupdate(deg1d.at[pl.ds(dbase, L)], onehot)
            for j in range(D // L):
                plsc.addupdate(agg_part.at[ld, pl.ds(j * L, L)],
                               rowbuf[p * GB + e, pl.ds(j * L, L)])
            return carry

        lax.fori_loop(0, n, edge, 0)

    def _fire(args):
        cur, nf = args
        par = nf & 1
        for t in range(GB // L):
            gidx[pl.ds(par * GB + t * L, L)] = csrc[pl.ds(t * L, L)]
            cldsave[pl.ds(par * GB + t * L, L)] = cld[pl.ds(t * L, L)]
        _start_gather(par)
        rs = csrc[pl.ds(GB, L)]
        rl = cld[pl.ds(GB, L)]
        csrc[pl.ds(0, L)] = rs
        cld[pl.ds(0, L)] = rl

        @pl.when(nf > 0)
        def _():
            _consume(1 - par, GB)

        return cur - GB, nf + 1

    def _grp(off, carry):
        cur, nf = carry
        dv = dsts_v[pl.ds(off, L)]
        ld = dv - my_base
        ok = (ld >= 0) & (ld < ROWS_PER_TILE)

        def _compact(args):
            cur, nf = args
            sv = srcs_v[pl.ds(off, L)]
            inc = plsc.cumsum(ok.astype(jnp.int32))
            pos = cur + inc - 1
            plsc.store_scatter(csrc, [pos], sv, mask=ok)
            plsc.store_scatter(cld, [pos], ld, mask=ok)
            cur = cur + inc[L - 1]
            return lax.cond(cur >= GB, _fire, lambda a: a, (cur, nf))

        return lax.cond(jnp.any(ok), _compact, lambda a: a, (cur, nf))

    def _start_scan(i, par):
        base = i * SK
        off = par * SK
        pltpu.async_copy(src_hbm.at[pl.ds(base, SK)],
                         srcs_v.at[pl.ds(off, SK)], sem1)
        pltpu.async_copy(dst_hbm.at[pl.ds(base, SK)],
                         dsts_v.at[pl.ds(off, SK)], sem2)

    def _wait_scan(i, par):
        base = i * SK
        off = par * SK
        pltpu.make_async_copy(src_hbm.at[pl.ds(base, SK)],
                              srcs_v.at[pl.ds(off, SK)], sem1).wait()
        pltpu.make_async_copy(dst_hbm.at[pl.ds(base, SK)],
                              dsts_v.at[pl.ds(off, SK)], sem2).wait()

    def _chunk(i, carry):
        par = i % 2
        _wait_scan(i, par)

        @pl.when(i + 1 < NCH)
        def _():
            _start_scan(i + 1, 1 - par)

        off = par * SK

        def _grp_off(g, cc):
            return _grp(off + g * L, cc)

        return lax.fori_loop(0, SK // L, _grp_off, carry)

    _start_scan(0, 0)
    cur, nf = lax.fori_loop(0, NCH, _chunk, (jnp.int32(0), jnp.int32(0)))

    @pl.when(nf > 0)
    def _():
        _consume((nf - 1) & 1, GB)

    par = nf & 1
    for t in range(GB // L):
        gidx[pl.ds(par * GB + t * L, L)] = csrc[pl.ds(t * L, L)]
        cldsave[pl.ds(par * GB + t * L, L)] = cld[pl.ds(t * L, L)]
    _start_gather(par)
    _consume(par, cur)

    @pl.loop(0, ROWS_PER_TILE)
    def _(r):
        invv = 1.0 / jnp.maximum(deg1d[pl.ds(r, L)], 1.0)
        inv = jnp.full((L,), invv[0], jnp.float32)
        for j in range(D // L):
            agg_part[r, pl.ds(j * L, L)] = agg_part[r, pl.ds(j * L, L)] * inv

    pltpu.sync_copy(agg_part, out_hbm.at[pl.ds(my_base, ROWS_PER_TILE)])


def _mm_body(a_ref, w_ref, b_ref, o_ref):
    acc = jnp.dot(a_ref[:, :], w_ref[:, :], preferred_element_type=jnp.float32)
    o_ref[:, :] = jnp.maximum(acc + b_ref[:, :], 0.0)


def _matmul(aggn, W, b2):
    return pl.pallas_call(
        _mm_body,
        grid=(NPAD // 256,),
        in_specs=[
            pl.BlockSpec((256, D), lambda i: (i, 0)),
            pl.BlockSpec((D, D), lambda i: (0, 0)),
            pl.BlockSpec((1, D), lambda i: (0, 0)),
        ],
        out_specs=pl.BlockSpec((256, D), lambda i: (i, 0)),
        out_shape=jax.ShapeDtypeStruct((NPAD, D), jnp.float32),
    )(aggn, W, b2)


@functools.partial(
    pl.kernel,
    out_type=(jax.ShapeDtypeStruct((PE,), jnp.float32),
              jax.ShapeDtypeStruct((PE,), jnp.float32)),
    mesh=_MESH,
    compiler_params=pltpu.CompilerParams(needs_layout_passes=False),
    scratch_types=[
        pltpu.VMEM((2 * CK, D), jnp.float32),
        pltpu.VMEM((2 * CK, D), jnp.float32),
        pltpu.VMEM((2 * CK,), jnp.int32),
        pltpu.VMEM((2 * CK,), jnp.int32),
        pltpu.VMEM((L, L), jnp.float32),
        pltpu.VMEM((CK,), jnp.float32),
        pltpu.SemaphoreType.DMA,
        pltpu.SemaphoreType.DMA,
    ],
)
def _score_kernel(z_hbm, ps_hbm, pd_hbm, ns_hbm, nd_hbm, pos_hbm, neg_hbm,
                  a2, b2, si_v, di_v, p_t, out_v, semi, semr):
    c = lax.axis_index("c")
    s = lax.axis_index("s")
    w = s * NC + c
    iota = lax.iota(jnp.int32, L)
    zeros16 = jnp.zeros((L,), jnp.float32)

    for si_hbm, di_hbm, o_hbm in ((ps_hbm, pd_hbm, pos_hbm),
                                  (ns_hbm, nd_hbm, neg_hbm)):

        def _fetch_idx(t, par):
            base = (w + t * NW) * CK
            off = par * CK
            pltpu.async_copy(si_hbm.at[pl.ds(base, CK)],
                             si_v.at[pl.ds(off, CK)], semi)
            pltpu.async_copy(di_hbm.at[pl.ds(base, CK)],
                             di_v.at[pl.ds(off, CK)], semi)

        def _wait_idx(t, par):
            base = (w + t * NW) * CK
            off = par * CK
            pltpu.make_async_copy(si_hbm.at[pl.ds(base, CK)],
                                  si_v.at[pl.ds(off, CK)], semi).wait()
            pltpu.make_async_copy(di_hbm.at[pl.ds(base, CK)],
                                  di_v.at[pl.ds(off, CK)], semi).wait()

        def _start_rows(par):
            off = par * CK
            pltpu.async_copy(z_hbm.at[si_v.at[pl.ds(off, CK)]],
                             a2.at[pl.ds(off, CK)], semr)
            pltpu.async_copy(z_hbm.at[di_v.at[pl.ds(off, CK)]],
                             b2.at[pl.ds(off, CK)], semr)

        def _wait_rows(par):
            off = par * CK
            pltpu.make_async_copy(z_hbm.at[si_v.at[pl.ds(off, CK)]],
                                  a2.at[pl.ds(off, CK)], semr).wait()
            pltpu.make_async_copy(z_hbm.at[di_v.at[pl.ds(off, CK)]],
                                  b2.at[pl.ds(off, CK)], semr).wait()

        _fetch_idx(0, 0)
        _wait_idx(0, 0)
        _start_rows(0)

        @pl.loop(0, CPT)
        def _(t):
            cid = w + t * NW

            @pl.when(cid < PC)
            def _():
                par = t % 2
                off = par * CK
                nxt = cid + NW

                @pl.when(nxt < PC)
                def _():
                    _fetch_idx(t + 1, 1 - par)
                    _wait_idx(t + 1, 1 - par)

                _wait_rows(par)

                @pl.when(nxt < PC)
                def _():
                    _start_rows(1 - par)

                @pl.loop(0, CK // L)
                def _(g):
                    row0 = off + g * L
                    for e in range(L):
                        acc = zeros16
                        for u in range(D // L):
                            acc = acc + (a2[row0 + e, pl.ds(u * L, L)] *
                                         b2[row0 + e, pl.ds(u * L, L)])
                        p_t[e, :] = acc
                    tot = zeros16
                    for j in range(L):
                        tot = tot + plsc.load_gather(
                            p_t, [iota, jnp.full((L,), j, jnp.int32)])
                    out_v[pl.ds(g * L, L)] = tot

                pltpu.sync_copy(out_v, o_hbm.at[pl.ds(cid * CK, CK)])


def kernel(x, edge_index, pos_edge_index, neg_edge_index, W, b):
    src = edge_index[0]
    dst = edge_index[1]
    aggn = _agg_kernel(x, src, dst)
    z = _matmul(aggn, W, b.reshape(1, D))
    pos, neg = _score_kernel(z, pos_edge_index[0], pos_edge_index[1],
                             neg_edge_index[0], neg_edge_index[1])
    return (pos, neg)


# speedup vs baseline: 1.1666x; 1.1666x over previous
import functools

import jax
import jax.numpy as jnp
from jax import lax
from jax.experimental import pallas as pl
from jax.experimental.pallas import tpu as pltpu
from jax.experimental.pallas import tpu_sc as plsc

N = 10000
D = 256
E = 160000
PE = 80000

NC = 2
NS = 16
L = 16
NW = NC * NS

NPAD = 10240
ROWS_PER_TILE = NPAD // NW

SK = 2000
NCH = E // SK
GB = 64
CBUF = 96

CK = 64
PC = PE // CK
CPT = (PC + NW - 1) // NW

_MESH = plsc.VectorSubcoreMesh(
    core_axis_name="c", subcore_axis_name="s", num_cores=NC, num_subcores=NS)


@functools.partial(
    pl.kernel,
    out_type=jax.ShapeDtypeStruct((NPAD, D), jnp.float32),
    mesh=_MESH,
    compiler_params=pltpu.CompilerParams(needs_layout_passes=False),
    scratch_types=[
        pltpu.VMEM((ROWS_PER_TILE, D), jnp.float32),
        pltpu.VMEM((ROWS_PER_TILE + 2 * L, ), jnp.float32),
        pltpu.VMEM((2 * GB, D), jnp.float32),
        pltpu.VMEM((2 * SK,), jnp.int32),
        pltpu.VMEM((2 * SK,), jnp.int32),
        pltpu.VMEM((CBUF,), jnp.int32),
        pltpu.VMEM((CBUF,), jnp.int32),
        pltpu.VMEM((2 * GB,), jnp.int32),
        pltpu.VMEM((2 * GB + L,), jnp.int32),
        pltpu.SemaphoreType.DMA,
        pltpu.SemaphoreType.DMA,
        pltpu.SemaphoreType.DMA,
    ],
)
def _agg_kernel(x_hbm, src_hbm, dst_hbm, out_hbm,
                agg_part, deg1d, rowbuf, srcs_v, dsts_v, csrc, cld, gidx,
                cldsave, sem1, sem2, semg):
    c = lax.axis_index("c")
    s = lax.axis_index("s")
    w = s * NC + c
    my_base = w * ROWS_PER_TILE

    zeros16f = jnp.zeros((L,), jnp.float32)
    zeros16i = jnp.zeros((L,), jnp.int32)
    iota = lax.iota(jnp.int32, L)

    @pl.loop(0, ROWS_PER_TILE)
    def _(r):
        for j in range(D // L):
            agg_part[r, pl.ds(j * L, L)] = zeros16f
    for t in range((ROWS_PER_TILE + 2 * L) // L):
        deg1d[pl.ds(t * L, L)] = zeros16f
    for t in range(CBUF // L):
        csrc[pl.ds(t * L, L)] = zeros16i

    def _start_gather(p):
        pltpu.async_copy(x_hbm.at[gidx.at[pl.ds(p * GB, GB)]],
                         rowbuf.at[pl.ds(p * GB, GB)], semg)

    def _consume(p, n):
        pltpu.make_async_copy(x_hbm.at[gidx.at[pl.ds(p * GB, GB)]],
                              rowbuf.at[pl.ds(p * GB, GB)], semg).wait()

        def edge(e, carry):
            ld = cldsave[pl.ds(p * GB + e, L)][0]
            dbase = ld & ~(L - 1)
            onehot = jnp.where(iota == ld - dbase, 1.0, 0.0)
            plsc.addupdate(deg1d.at[pl.ds(dbase, L)], onehot)
            for j in range(D // L):
                plsc.addupdate(agg_part.at[ld, pl.ds(j * L, L)],
                               rowbuf[p * GB + e, pl.ds(j * L, L)])
            return carry

        lax.fori_loop(0, n, edge, 0)

    def _fire(args):
        cur, nf = args
        par = nf & 1
        for t in range(GB // L):
            gidx[pl.ds(par * GB + t * L, L)] = csrc[pl.ds(t * L, L)]
            cldsave[pl.ds(par * GB + t * L, L)] = cld[pl.ds(t * L, L)]
        _start_gather(par)
        rs = csrc[pl.ds(GB, L)]
        rl = cld[pl.ds(GB, L)]
        csrc[pl.ds(0, L)] = rs
        cld[pl.ds(0, L)] = rl

        @pl.when(nf > 0)
        def _():
            _consume(1 - par, GB)

        return cur - GB, nf + 1

    def _grp(off, carry):
        cur, nf = carry
        dv = dsts_v[pl.ds(off, L)]
        sv = srcs_v[pl.ds(off, L)]
        ld = dv - my_base
        ok = (ld >= 0) & (ld < ROWS_PER_TILE)
        inc = plsc.cumsum(ok.astype(jnp.int32))
        pos = cur + inc - 1
        plsc.store_scatter(csrc, [pos], sv, mask=ok)
        plsc.store_scatter(cld, [pos], ld, mask=ok)
        cur = cur + inc[L - 1]
        return lax.cond(cur >= GB, _fire, lambda a: a, (cur, nf))

    def _start_scan(i, par):
        base = i * SK
        off = par * SK
        pltpu.async_copy(src_hbm.at[pl.ds(base, SK)],
                         srcs_v.at[pl.ds(off, SK)], sem1)
        pltpu.async_copy(dst_hbm.at[pl.ds(base, SK)],
                         dsts_v.at[pl.ds(off, SK)], sem2)

    def _wait_scan(i, par):
        base = i * SK
        off = par * SK
        pltpu.make_async_copy(src_hbm.at[pl.ds(base, SK)],
                              srcs_v.at[pl.ds(off, SK)], sem1).wait()
        pltpu.make_async_copy(dst_hbm.at[pl.ds(base, SK)],
                              dsts_v.at[pl.ds(off, SK)], sem2).wait()

    def _chunk(i, carry):
        par = i % 2
        _wait_scan(i, par)

        @pl.when(i + 1 < NCH)
        def _():
            _start_scan(i + 1, 1 - par)

        off = par * SK

        def _grp_off(g, cc):
            return _grp(off + g * L, cc)

        return lax.fori_loop(0, SK // L, _grp_off, carry)

    _start_scan(0, 0)
    cur, nf = lax.fori_loop(0, NCH, _chunk, (jnp.int32(0), jnp.int32(0)))

    @pl.when(nf > 0)
    def _():
        _consume((nf - 1) & 1, GB)

    par = nf & 1
    for t in range(GB // L):
        gidx[pl.ds(par * GB + t * L, L)] = csrc[pl.ds(t * L, L)]
        cldsave[pl.ds(par * GB + t * L, L)] = cld[pl.ds(t * L, L)]
    _start_gather(par)
    _consume(par, cur)

    @pl.loop(0, ROWS_PER_TILE)
    def _(r):
        invv = 1.0 / jnp.maximum(deg1d[pl.ds(r, L)], 1.0)
        inv = jnp.full((L,), invv[0], jnp.float32)
        for j in range(D // L):
            agg_part[r, pl.ds(j * L, L)] = agg_part[r, pl.ds(j * L, L)] * inv

    pltpu.sync_copy(agg_part, out_hbm.at[pl.ds(my_base, ROWS_PER_TILE)])


def _mm_body(a_ref, w_ref, b_ref, o_ref):
    acc = jnp.dot(a_ref[:, :], w_ref[:, :], preferred_element_type=jnp.float32)
    o_ref[:, :] = jnp.maximum(acc + b_ref[:, :], 0.0)


def _matmul(aggn, W, b2):
    return pl.pallas_call(
        _mm_body,
        grid=(NPAD // 256,),
        in_specs=[
            pl.BlockSpec((256, D), lambda i: (i, 0)),
            pl.BlockSpec((D, D), lambda i: (0, 0)),
            pl.BlockSpec((1, D), lambda i: (0, 0)),
        ],
        out_specs=pl.BlockSpec((256, D), lambda i: (i, 0)),
        out_shape=jax.ShapeDtypeStruct((NPAD, D), jnp.float32),
    )(aggn, W, b2)


@functools.partial(
    pl.kernel,
    out_type=(jax.ShapeDtypeStruct((PE,), jnp.float32),
              jax.ShapeDtypeStruct((PE,), jnp.float32)),
    mesh=_MESH,
    compiler_params=pltpu.CompilerParams(needs_layout_passes=False),
    scratch_types=[
        pltpu.VMEM((2 * CK, D), jnp.float32),
        pltpu.VMEM((2 * CK, D), jnp.float32),
        pltpu.VMEM((2 * CK,), jnp.int32),
        pltpu.VMEM((2 * CK,), jnp.int32),
        pltpu.VMEM((L, L), jnp.float32),
        pltpu.VMEM((CK,), jnp.float32),
        pltpu.SemaphoreType.DMA,
        pltpu.SemaphoreType.DMA,
    ],
)
def _score_kernel(z_hbm, ps_hbm, pd_hbm, ns_hbm, nd_hbm, pos_hbm, neg_hbm,
                  a2, b2, si_v, di_v, p_t, out_v, semi, semr):
    c = lax.axis_index("c")
    s = lax.axis_index("s")
    w = s * NC + c
    iota = lax.iota(jnp.int32, L)
    zeros16 = jnp.zeros((L,), jnp.float32)

    for si_hbm, di_hbm, o_hbm in ((ps_hbm, pd_hbm, pos_hbm),
                                  (ns_hbm, nd_hbm, neg_hbm)):

        def _fetch_idx(t, par):
            base = (w + t * NW) * CK
            off = par * CK
            pltpu.async_copy(si_hbm.at[pl.ds(base, CK)],
                             si_v.at[pl.ds(off, CK)], semi)
            pltpu.async_copy(di_hbm.at[pl.ds(base, CK)],
                             di_v.at[pl.ds(off, CK)], semi)

        def _wait_idx(t, par):
            base = (w + t * NW) * CK
            off = par * CK
            pltpu.make_async_copy(si_hbm.at[pl.ds(base, CK)],
                                  si_v.at[pl.ds(off, CK)], semi).wait()
            pltpu.make_async_copy(di_hbm.at[pl.ds(base, CK)],
                                  di_v.at[pl.ds(off, CK)], semi).wait()

        def _start_rows(par):
            off = par * CK
            pltpu.async_copy(z_hbm.at[si_v.at[pl.ds(off, CK)]],
                             a2.at[pl.ds(off, CK)], semr)
            pltpu.async_copy(z_hbm.at[di_v.at[pl.ds(off, CK)]],
                             b2.at[pl.ds(off, CK)], semr)

        def _wait_rows(par):
            off = par * CK
            pltpu.make_async_copy(z_hbm.at[si_v.at[pl.ds(off, CK)]],
                                  a2.at[pl.ds(off, CK)], semr).wait()
            pltpu.make_async_copy(z_hbm.at[di_v.at[pl.ds(off, CK)]],
                                  b2.at[pl.ds(off, CK)], semr).wait()

        _fetch_idx(0, 0)
        _wait_idx(0, 0)
        _start_rows(0)

        @pl.loop(0, CPT)
        def _(t):
            cid = w + t * NW

            @pl.when(cid < PC)
            def _():
                par = t % 2
                off = par * CK
                nxt = cid + NW

                @pl.when(nxt < PC)
                def _():
                    _fetch_idx(t + 1, 1 - par)
                    _wait_idx(t + 1, 1 - par)

                _wait_rows(par)

                @pl.when(nxt < PC)
                def _():
                    _start_rows(1 - par)

                @pl.loop(0, CK // L)
                def _(g):
                    row0 = off + g * L
                    for e in range(L):
                        acc = zeros16
                        for u in range(D // L):
                            acc = acc + (a2[row0 + e, pl.ds(u * L, L)] *
                                         b2[row0 + e, pl.ds(u * L, L)])
                        p_t[e, :] = acc
                    tot = zeros16
                    for j in range(L):
                        tot = tot + plsc.load_gather(
                            p_t, [iota, jnp.full((L,), j, jnp.int32)])
                    out_v[pl.ds(g * L, L)] = tot

                pltpu.sync_copy(out_v, o_hbm.at[pl.ds(cid * CK, CK)])


def kernel(x, edge_index, pos_edge_index, neg_edge_index, W, b):
    src = edge_index[0]
    dst = edge_index[1]
    aggn = _agg_kernel(x, src, dst)
    z = _matmul(aggn, W, b.reshape(1, D))
    pos, neg = _score_kernel(z, pos_edge_index[0], pos_edge_index[1],
                             neg_edge_index[0], neg_edge_index[1])
    return (pos, neg)


# speedup vs baseline: 1.3347x; 1.1441x over previous
import functools

import jax
import jax.numpy as jnp
from jax import lax
from jax.experimental import pallas as pl
from jax.experimental.pallas import tpu as pltpu
from jax.experimental.pallas import tpu_sc as plsc

N = 10000
D = 256
E = 160000
PE = 80000

NC = 2
NS = 16
L = 16
NW = NC * NS

NPAD = 10240
ROWS_PER_TILE = NPAD // NW

SK = 1600
NCH = E // SK
GB = 64
CBUF = 144
GPC = 4

CK = 64
PC = PE // CK
CPT = (PC + NW - 1) // NW

_MESH = plsc.VectorSubcoreMesh(
    core_axis_name="c", subcore_axis_name="s", num_cores=NC, num_subcores=NS)


def _vbcast(x, lane):
    idx = jnp.full((L, 1), lane, jnp.int32)
    dnums = lax.GatherDimensionNumbers(
        offset_dims=(), collapsed_slice_dims=(0,), start_index_map=(0,))
    return lax.gather(x, idx, dnums, (1,),
                      mode=lax.GatherScatterMode.PROMISE_IN_BOUNDS)


@functools.partial(
    pl.kernel,
    out_type=jax.ShapeDtypeStruct((NPAD * D,), jnp.float32),
    mesh=_MESH,
    compiler_params=pltpu.CompilerParams(needs_layout_passes=False),
    scratch_types=[
        pltpu.VMEM((ROWS_PER_TILE * D,), jnp.float32),
        pltpu.VMEM((ROWS_PER_TILE + 2 * L, ), jnp.float32),
        pltpu.VMEM((2 * GB, D), jnp.float32),
        pltpu.VMEM((2 * SK,), jnp.int32),
        pltpu.VMEM((2 * SK,), jnp.int32),
        pltpu.VMEM((CBUF,), jnp.int32),
        pltpu.VMEM((CBUF,), jnp.int32),
        pltpu.VMEM((2 * GB,), jnp.int32),
        pltpu.VMEM((2 * GB + L,), jnp.int32),
        pltpu.SemaphoreType.DMA,
        pltpu.SemaphoreType.DMA,
        pltpu.SemaphoreType.DMA,
    ],
)
def _agg_kernel(x_hbm, src_hbm, dst_hbm, out_hbm,
                agg_flat, deg1d, rowbuf, srcs_v, dsts_v, csrc, cld, gidx,
                cldsave, sem1, sem2, semg):
    c = lax.axis_index("c")
    s = lax.axis_index("s")
    w = s * NC + c
    my_base = w * ROWS_PER_TILE

    zeros16f = jnp.zeros((L,), jnp.float32)
    zeros16i = jnp.zeros((L,), jnp.int32)
    ones16 = jnp.ones((L,), jnp.float32)
    iota = lax.iota(jnp.int32, L)
    mask0 = iota == 0

    @pl.loop(0, ROWS_PER_TILE)
    def _(r):
        for j in range(D // L):
            agg_flat[pl.ds(r * D + j * L, L)] = zeros16f
    for t in range((ROWS_PER_TILE + 2 * L) // L):
        deg1d[pl.ds(t * L, L)] = zeros16f
    for t in range(CBUF // L):
        csrc[pl.ds(t * L, L)] = zeros16i

    def _start_gather(p):
        pltpu.async_copy(x_hbm.at[gidx.at[pl.ds(p * GB, GB)]],
                         rowbuf.at[pl.ds(p * GB, GB)], semg)

    def _consume(p, n):
        pltpu.make_async_copy(x_hbm.at[gidx.at[pl.ds(p * GB, GB)]],
                              rowbuf.at[pl.ds(p * GB, GB)], semg).wait()

        def edge(e, carry):
            cldv = cldsave[pl.ds(p * GB + e, L)]
            ldvec = _vbcast(cldv, 0)
            base = ldvec << 8
            plsc.addupdate_scatter(deg1d, [ldvec], ones16, mask=mask0)
            for j in range(D // L):
                plsc.addupdate_scatter(agg_flat, [base + (iota + j * L)],
                                       rowbuf[p * GB + e, pl.ds(j * L, L)])
            return carry

        lax.fori_loop(0, n, edge, 0)

    def _fire(args):
        cur_v, nf = args
        par = nf & 1
        for t in range(GB // L):
            gidx[pl.ds(par * GB + t * L, L)] = csrc[pl.ds(t * L, L)]
            cldsave[pl.ds(par * GB + t * L, L)] = cld[pl.ds(t * L, L)]
        _start_gather(par)
        for t in range(GB // L):
            rs = csrc[pl.ds(GB + t * L, L)]
            rl = cld[pl.ds(GB + t * L, L)]
            csrc[pl.ds(t * L, L)] = rs
            cld[pl.ds(t * L, L)] = rl

        @pl.when(nf > 0)
        def _():
            _consume(1 - par, GB)

        return cur_v - GB, nf + 1

    def _ckpt(off):
        def ckpt(q, carry):
            cur_v, nf = carry
            for qq in range(GPC):
                goff = off + q * (GPC * L) + qq * L
                dv = dsts_v[pl.ds(goff, L)]
                sv = srcs_v[pl.ds(goff, L)]
                ld = dv - my_base
                ok = (ld >= 0) & (ld < ROWS_PER_TILE)
                inc = plsc.cumsum(ok.astype(jnp.int32))
                pos = cur_v + inc - 1
                plsc.store_scatter(csrc, [pos], sv, mask=ok)
                plsc.store_scatter(cld, [pos], ld, mask=ok)
                cur_v = cur_v + _vbcast(inc, L - 1)
            return lax.cond(cur_v[0] >= GB, _fire, lambda a: a, (cur_v, nf))

        return ckpt

    def _start_scan(i, par):
        base = i * SK
        off = par * SK
        pltpu.async_copy(src_hbm.at[pl.ds(base, SK)],
                         srcs_v.at[pl.ds(off, SK)], sem1)
        pltpu.async_copy(dst_hbm.at[pl.ds(base, SK)],
                         dsts_v.at[pl.ds(off, SK)], sem2)

    def _wait_scan(i, par):
        base = i * SK
        off = par * SK
        pltpu.make_async_copy(src_hbm.at[pl.ds(base, SK)],
                              srcs_v.at[pl.ds(off, SK)], sem1).wait()
        pltpu.make_async_copy(dst_hbm.at[pl.ds(base, SK)],
                              dsts_v.at[pl.ds(off, SK)], sem2).wait()

    def _chunk(i, carry):
        par = i % 2
        _wait_scan(i, par)

        @pl.when(i + 1 < NCH)
        def _():
            _start_scan(i + 1, 1 - par)

        off = par * SK
        return lax.fori_loop(0, SK // (GPC * L), _ckpt(off), carry)

    _start_scan(0, 0)
    cur_v, nf = lax.fori_loop(0, NCH, _chunk,
                              (jnp.zeros((L,), jnp.int32), jnp.int32(0)))
    cur = cur_v[0]

    @pl.when(nf > 0)
    def _():
        _consume((nf - 1) & 1, GB)

    par = nf & 1
    for t in range(GB // L):
        gidx[pl.ds(par * GB + t * L, L)] = csrc[pl.ds(t * L, L)]
        cldsave[pl.ds(par * GB + t * L, L)] = cld[pl.ds(t * L, L)]
    _start_gather(par)
    _consume(par, cur)

    @pl.loop(0, ROWS_PER_TILE)
    def _(r):
        invv = 1.0 / jnp.maximum(deg1d[pl.ds(r, L)], 1.0)
        inv = _vbcast(invv, 0)
        for j in range(D // L):
            o = r * D + j * L
            agg_flat[pl.ds(o, L)] = agg_flat[pl.ds(o, L)] * inv

    pltpu.sync_copy(agg_flat,
                    out_hbm.at[pl.ds(my_base * D, ROWS_PER_TILE * D)])


def _mm_body(a_ref, w_ref, b_ref, o_ref):
    acc = jnp.dot(a_ref[:, :], w_ref[:, :], preferred_element_type=jnp.float32)
    o_ref[:, :] = jnp.maximum(acc + b_ref[:, :], 0.0)


def _matmul(aggn, W, b2):
    return pl.pallas_call(
        _mm_body,
        grid=(NPAD // 256,),
        in_specs=[
            pl.BlockSpec((256, D), lambda i: (i, 0)),
            pl.BlockSpec((D, D), lambda i: (0, 0)),
            pl.BlockSpec((1, D), lambda i: (0, 0)),
        ],
        out_specs=pl.BlockSpec((256, D), lambda i: (i, 0)),
        out_shape=jax.ShapeDtypeStruct((NPAD, D), jnp.float32),
    )(aggn, W, b2)


@functools.partial(
    pl.kernel,
    out_type=(jax.ShapeDtypeStruct((PE,), jnp.float32),
              jax.ShapeDtypeStruct((PE,), jnp.float32)),
    mesh=_MESH,
    compiler_params=pltpu.CompilerParams(needs_layout_passes=False),
    scratch_types=[
        pltpu.VMEM((2 * CK, D), jnp.float32),
        pltpu.VMEM((2 * CK, D), jnp.float32),
        pltpu.VMEM((2 * CK,), jnp.int32),
        pltpu.VMEM((2 * CK,), jnp.int32),
        pltpu.VMEM((L, L), jnp.float32),
        pltpu.VMEM((CK,), jnp.float32),
        pltpu.SemaphoreType.DMA,
        pltpu.SemaphoreType.DMA,
    ],
)
def _score_kernel(z_hbm, ps_hbm, pd_hbm, ns_hbm, nd_hbm, pos_hbm, neg_hbm,
                  a2, b2, si_v, di_v, p_t, out_v, semi, semr):
    c = lax.axis_index("c")
    s = lax.axis_index("s")
    w = s * NC + c
    iota = lax.iota(jnp.int32, L)
    zeros16 = jnp.zeros((L,), jnp.float32)

    for si_hbm, di_hbm, o_hbm in ((ps_hbm, pd_hbm, pos_hbm),
                                  (ns_hbm, nd_hbm, neg_hbm)):

        def _fetch_idx(t, par):
            base = (w + t * NW) * CK
            off = par * CK
            pltpu.async_copy(si_hbm.at[pl.ds(base, CK)],
                             si_v.at[pl.ds(off, CK)], semi)
            pltpu.async_copy(di_hbm.at[pl.ds(base, CK)],
                             di_v.at[pl.ds(off, CK)], semi)

        def _wait_idx(t, par):
            base = (w + t * NW) * CK
            off = par * CK
            pltpu.make_async_copy(si_hbm.at[pl.ds(base, CK)],
                                  si_v.at[pl.ds(off, CK)], semi).wait()
            pltpu.make_async_copy(di_hbm.at[pl.ds(base, CK)],
                                  di_v.at[pl.ds(off, CK)], semi).wait()

        def _start_rows(par):
            off = par * CK
            pltpu.async_copy(z_hbm.at[si_v.at[pl.ds(off, CK)]],
                             a2.at[pl.ds(off, CK)], semr)
            pltpu.async_copy(z_hbm.at[di_v.at[pl.ds(off, CK)]],
                             b2.at[pl.ds(off, CK)], semr)

        def _wait_rows(par):
            off = par * CK
            pltpu.make_async_copy(z_hbm.at[si_v.at[pl.ds(off, CK)]],
                                  a2.at[pl.ds(off, CK)], semr).wait()
            pltpu.make_async_copy(z_hbm.at[di_v.at[pl.ds(off, CK)]],
                                  b2.at[pl.ds(off, CK)], semr).wait()

        _fetch_idx(0, 0)
        _wait_idx(0, 0)
        _start_rows(0)

        @pl.loop(0, CPT)
        def _(t):
            cid = w + t * NW

            @pl.when(cid < PC)
            def _():
                par = t % 2
                off = par * CK
                nxt = cid + NW

                @pl.when(nxt < PC)
                def _():
                    _fetch_idx(t + 1, 1 - par)
                    _wait_idx(t + 1, 1 - par)

                _wait_rows(par)

                @pl.when(nxt < PC)
                def _():
                    _start_rows(1 - par)

                @pl.loop(0, CK // L)
                def _(g):
                    row0 = off + g * L
                    for e in range(L):
                        acc = zeros16
                        for u in range(D // L):
                            acc = acc + (a2[row0 + e, pl.ds(u * L, L)] *
                                         b2[row0 + e, pl.ds(u * L, L)])
                        p_t[e, :] = acc
                    tot = zeros16
                    for j in range(L):
                        tot = tot + plsc.load_gather(
                            p_t, [iota, jnp.full((L,), j, jnp.int32)])
                    out_v[pl.ds(g * L, L)] = tot

                pltpu.sync_copy(out_v, o_hbm.at[pl.ds(cid * CK, CK)])


def kernel(x, edge_index, pos_edge_index, neg_edge_index, W, b):
    src = edge_index[0]
    dst = edge_index[1]
    aggn = _agg_kernel(x, src, dst).reshape(NPAD, D)
    z = _matmul(aggn, W, b.reshape(1, D))
    pos, neg = _score_kernel(z, pos_edge_index[0], pos_edge_index[1],
                             neg_edge_index[0], neg_edge_index[1])
    return (pos, neg)


# speedup vs baseline: 2.1306x; 1.5964x over previous
import functools

import jax
import jax.numpy as jnp
from jax import lax
from jax.experimental import pallas as pl
from jax.experimental.pallas import tpu as pltpu
from jax.experimental.pallas import tpu_sc as plsc

N = 10000
D = 256
E = 160000
PE = 80000

NC = 2
NS = 16
L = 16
NW = NC * NS

NPAD = 10240
ROWS_PER_TILE = NPAD // NW

SK = 1600
NCH = E // SK
GB = 64
CBUF = 144
GPC = 4

CK = 64
PC = PE // CK
CPT = (PC + NW - 1) // NW

_MESH = plsc.VectorSubcoreMesh(
    core_axis_name="c", subcore_axis_name="s", num_cores=NC, num_subcores=NS)


def _vbcast(x, lane):
    idx = jnp.full((L, 1), lane, jnp.int32)
    dnums = lax.GatherDimensionNumbers(
        offset_dims=(), collapsed_slice_dims=(0,), start_index_map=(0,))
    return lax.gather(x, idx, dnums, (1,),
                      mode=lax.GatherScatterMode.PROMISE_IN_BOUNDS)


@functools.partial(
    pl.kernel,
    out_type=jax.ShapeDtypeStruct((NPAD * D,), jnp.float32),
    mesh=_MESH,
    compiler_params=pltpu.CompilerParams(needs_layout_passes=False),
    scratch_types=[
        pltpu.VMEM((ROWS_PER_TILE * D,), jnp.float32),
        pltpu.VMEM((ROWS_PER_TILE + 2 * L, ), jnp.float32),
        pltpu.VMEM((2 * GB, D), jnp.float32),
        pltpu.VMEM((2 * SK,), jnp.int32),
        pltpu.VMEM((2 * SK,), jnp.int32),
        pltpu.VMEM((CBUF,), jnp.int32),
        pltpu.VMEM((CBUF,), jnp.int32),
        pltpu.VMEM((2 * GB,), jnp.int32),
        pltpu.VMEM((2 * GB + L,), jnp.int32),
        pltpu.SemaphoreType.DMA,
        pltpu.SemaphoreType.DMA,
        pltpu.SemaphoreType.DMA,
    ],
)
def _agg_kernel(x_hbm, src_hbm, dst_hbm, out_hbm,
                agg_flat, deg1d, rowbuf, srcs_v, dsts_v, csrc, cld, gidx,
                cldsave, sem1, sem2, semg):
    c = lax.axis_index("c")
    s = lax.axis_index("s")
    w = s * NC + c
    my_base = w * ROWS_PER_TILE

    zeros16f = jnp.zeros((L,), jnp.float32)
    zeros16i = jnp.zeros((L,), jnp.int32)
    ones16 = jnp.ones((L,), jnp.float32)
    iota = lax.iota(jnp.int32, L)
    mask0 = iota == 0

    @pl.loop(0, ROWS_PER_TILE)
    def _(r):
        for j in range(D // L):
            agg_flat[pl.ds(r * D + j * L, L)] = zeros16f
    for t in range((ROWS_PER_TILE + 2 * L) // L):
        deg1d[pl.ds(t * L, L)] = zeros16f
    for t in range(CBUF // L):
        csrc[pl.ds(t * L, L)] = zeros16i

    def _start_gather(p):
        pltpu.async_copy(x_hbm.at[gidx.at[pl.ds(p * GB, GB)]],
                         rowbuf.at[pl.ds(p * GB, GB)], semg)

    def _consume(p, n):
        pltpu.make_async_copy(x_hbm.at[gidx.at[pl.ds(p * GB, GB)]],
                              rowbuf.at[pl.ds(p * GB, GB)], semg).wait()

        def edge(e, carry):
            cldv = cldsave[pl.ds(p * GB + e, L)]
            ldvec = _vbcast(cldv, 0)
            base = ldvec << 8
            vals = [rowbuf[p * GB + e, pl.ds(j * L, L)]
                    for j in range(D // L)]
            plsc.addupdate_scatter(deg1d, [ldvec], ones16, mask=mask0)
            for j in range(D // L):
                plsc.addupdate_scatter(agg_flat, [base + (iota + j * L)],
                                       vals[j])
            return carry

        lax.fori_loop(0, n, edge, 0)

    def _fire(args):
        cur_v, nf = args
        par = nf & 1
        for t in range(GB // L):
            gidx[pl.ds(par * GB + t * L, L)] = csrc[pl.ds(t * L, L)]
            cldsave[pl.ds(par * GB + t * L, L)] = cld[pl.ds(t * L, L)]
        _start_gather(par)
        for t in range(GB // L):
            rs = csrc[pl.ds(GB + t * L, L)]
            rl = cld[pl.ds(GB + t * L, L)]
            csrc[pl.ds(t * L, L)] = rs
            cld[pl.ds(t * L, L)] = rl

        @pl.when(nf > 0)
        def _():
            _consume(1 - par, GB)

        return cur_v - GB, nf + 1

    def _ckpt(off):
        def ckpt(q, carry):
            cur_v, nf = carry
            base_off = off + q * (GPC * L)
            dvs = [dsts_v[pl.ds(base_off + qq * L, L)] for qq in range(GPC)]
            svs = [srcs_v[pl.ds(base_off + qq * L, L)] for qq in range(GPC)]
            for qq in range(GPC):
                ld = dvs[qq] - my_base
                ok = (ld >= 0) & (ld < ROWS_PER_TILE)
                inc = plsc.cumsum(ok.astype(jnp.int32))
                pos = cur_v + inc - 1
                plsc.store_scatter(csrc, [pos], svs[qq], mask=ok)
                plsc.store_scatter(cld, [pos], ld, mask=ok)
                cur_v = cur_v + _vbcast(inc, L - 1)
            return lax.cond(cur_v[0] >= GB, _fire, lambda a: a, (cur_v, nf))

        return ckpt

    def _start_scan(i, par):
        base = i * SK
        off = par * SK
        pltpu.async_copy(src_hbm.at[pl.ds(base, SK)],
                         srcs_v.at[pl.ds(off, SK)], sem1)
        pltpu.async_copy(dst_hbm.at[pl.ds(base, SK)],
                         dsts_v.at[pl.ds(off, SK)], sem2)

    def _wait_scan(i, par):
        base = i * SK
        off = par * SK
        pltpu.make_async_copy(src_hbm.at[pl.ds(base, SK)],
                              srcs_v.at[pl.ds(off, SK)], sem1).wait()
        pltpu.make_async_copy(dst_hbm.at[pl.ds(base, SK)],
                              dsts_v.at[pl.ds(off, SK)], sem2).wait()

    def _chunk(i, carry):
        par = i % 2
        _wait_scan(i, par)

        @pl.when(i + 1 < NCH)
        def _():
            _start_scan(i + 1, 1 - par)

        off = par * SK
        return lax.fori_loop(0, SK // (GPC * L), _ckpt(off), carry)

    _start_scan(0, 0)
    cur_v, nf = lax.fori_loop(0, NCH, _chunk,
                              (jnp.zeros((L,), jnp.int32), jnp.int32(0)))
    cur = cur_v[0]

    @pl.when(nf > 0)
    def _():
        _consume((nf - 1) & 1, GB)

    par = nf & 1
    for t in range(GB // L):
        gidx[pl.ds(par * GB + t * L, L)] = csrc[pl.ds(t * L, L)]
        cldsave[pl.ds(par * GB + t * L, L)] = cld[pl.ds(t * L, L)]
    _start_gather(par)
    _consume(par, cur)

    @pl.loop(0, ROWS_PER_TILE)
    def _(r):
        invv = 1.0 / jnp.maximum(deg1d[pl.ds(r, L)], 1.0)
        inv = _vbcast(invv, 0)
        for j in range(D // L):
            o = r * D + j * L
            agg_flat[pl.ds(o, L)] = agg_flat[pl.ds(o, L)] * inv

    pltpu.sync_copy(agg_flat,
                    out_hbm.at[pl.ds(my_base * D, ROWS_PER_TILE * D)])


def _mm_body(a_ref, w_ref, b_ref, o_ref):
    acc = jnp.dot(a_ref[:, :], w_ref[:, :], preferred_element_type=jnp.float32)
    o_ref[:, :] = jnp.maximum(acc + b_ref[:, :], 0.0)


def _matmul(aggn, W, b2):
    return pl.pallas_call(
        _mm_body,
        grid=(NPAD // 256,),
        in_specs=[
            pl.BlockSpec((256, D), lambda i: (i, 0)),
            pl.BlockSpec((D, D), lambda i: (0, 0)),
            pl.BlockSpec((1, D), lambda i: (0, 0)),
        ],
        out_specs=pl.BlockSpec((256, D), lambda i: (i, 0)),
        out_shape=jax.ShapeDtypeStruct((NPAD, D), jnp.float32),
    )(aggn, W, b2)


@functools.partial(
    pl.kernel,
    out_type=(jax.ShapeDtypeStruct((PE,), jnp.float32),
              jax.ShapeDtypeStruct((PE,), jnp.float32)),
    mesh=_MESH,
    compiler_params=pltpu.CompilerParams(needs_layout_passes=False),
    scratch_types=[
        pltpu.VMEM((2 * CK, D), jnp.float32),
        pltpu.VMEM((2 * CK, D), jnp.float32),
        pltpu.VMEM((2 * CK,), jnp.int32),
        pltpu.VMEM((2 * CK,), jnp.int32),
        pltpu.VMEM((L, L), jnp.float32),
        pltpu.VMEM((CK,), jnp.float32),
        pltpu.SemaphoreType.DMA,
        pltpu.SemaphoreType.DMA,
    ],
)
def _score_kernel(z_hbm, ps_hbm, pd_hbm, ns_hbm, nd_hbm, pos_hbm, neg_hbm,
                  a2, b2, si_v, di_v, p_t, out_v, semi, semr):
    c = lax.axis_index("c")
    s = lax.axis_index("s")
    w = s * NC + c
    iota = lax.iota(jnp.int32, L)
    zeros16 = jnp.zeros((L,), jnp.float32)

    for si_hbm, di_hbm, o_hbm in ((ps_hbm, pd_hbm, pos_hbm),
                                  (ns_hbm, nd_hbm, neg_hbm)):

        def _fetch_idx(t, par):
            base = (w + t * NW) * CK
            off = par * CK
            pltpu.async_copy(si_hbm.at[pl.ds(base, CK)],
                             si_v.at[pl.ds(off, CK)], semi)
            pltpu.async_copy(di_hbm.at[pl.ds(base, CK)],
                             di_v.at[pl.ds(off, CK)], semi)

        def _wait_idx(t, par):
            base = (w + t * NW) * CK
            off = par * CK
            pltpu.make_async_copy(si_hbm.at[pl.ds(base, CK)],
                                  si_v.at[pl.ds(off, CK)], semi).wait()
            pltpu.make_async_copy(di_hbm.at[pl.ds(base, CK)],
                                  di_v.at[pl.ds(off, CK)], semi).wait()

        def _start_rows(par):
            off = par * CK
            pltpu.async_copy(z_hbm.at[si_v.at[pl.ds(off, CK)]],
                             a2.at[pl.ds(off, CK)], semr)
            pltpu.async_copy(z_hbm.at[di_v.at[pl.ds(off, CK)]],
                             b2.at[pl.ds(off, CK)], semr)

        def _wait_rows(par):
            off = par * CK
            pltpu.make_async_copy(z_hbm.at[si_v.at[pl.ds(off, CK)]],
                                  a2.at[pl.ds(off, CK)], semr).wait()
            pltpu.make_async_copy(z_hbm.at[di_v.at[pl.ds(off, CK)]],
                                  b2.at[pl.ds(off, CK)], semr).wait()

        _fetch_idx(0, 0)
        _wait_idx(0, 0)
        _start_rows(0)

        @pl.loop(0, CPT)
        def _(t):
            cid = w + t * NW

            @pl.when(cid < PC)
            def _():
                par = t % 2
                off = par * CK
                nxt = cid + NW

                @pl.when(nxt < PC)
                def _():
                    _fetch_idx(t + 1, 1 - par)
                    _wait_idx(t + 1, 1 - par)

                _wait_rows(par)

                @pl.when(nxt < PC)
                def _():
                    _start_rows(1 - par)

                @pl.loop(0, CK // L)
                def _(g):
                    row0 = off + g * L
                    for e in range(L):
                        acc = zeros16
                        for u in range(D // L):
                            acc = acc + (a2[row0 + e, pl.ds(u * L, L)] *
                                         b2[row0 + e, pl.ds(u * L, L)])
                        p_t[e, :] = acc
                    tot = zeros16
                    for j in range(L):
                        tot = tot + plsc.load_gather(
                            p_t, [iota, jnp.full((L,), j, jnp.int32)])
                    out_v[pl.ds(g * L, L)] = tot

                pltpu.sync_copy(out_v, o_hbm.at[pl.ds(cid * CK, CK)])


def kernel(x, edge_index, pos_edge_index, neg_edge_index, W, b):
    src = edge_index[0]
    dst = edge_index[1]
    aggn = _agg_kernel(x, src, dst).reshape(NPAD, D)
    z = _matmul(aggn, W, b.reshape(1, D))
    pos, neg = _score_kernel(z, pos_edge_index[0], pos_edge_index[1],
                             neg_edge_index[0], neg_edge_index[1])
    return (pos, neg)
